# Initial kernel scaffold; baseline (speedup 1.0000x reference)
import jax
import jax.numpy as jnp
from jax.experimental import pallas as pl


def kernel(h, lattice_flat, non_zscored_lattice, frac_diff, num_atoms_one_hot, ln_gamma, ln_beta, W_na, W_e1, b_e1, W_e2, b_e2, W_n1, b_n1, W_n2, b_n2, edge_index, edge2graph):
    raise NotImplementedError("write your pallas kernel here")


# speedup vs baseline: 2.4823x; 2.4823x over previous
import functools
import math

import jax
import jax.numpy as jnp
from jax import lax
from jax.experimental import pallas as pl
from jax.experimental.pallas import tpu as pltpu
from jax.experimental.pallas import tpu_sc as plsc

_N = 10000
_E = 160000
_G = 256
_H = 128
_NF = 16
_DL = 6
_MA = 100

_BN = 1000
_BE = 1280
_WPAD = 144

_NC = 2
_NS = 16
_NW = _NC * _NS
_GCH = 80
_GROWS = 2 * _E // _NW
_GCHUNKS = _GROWS // _GCH
_SCH = 128
_SCHUNKS = _E // _SCH
_ZB = 40


def _prologue_body(h_ref, par_ref, whi_ref, whj_ref, hn_ref, p2_ref):
    x = h_ref[...]
    mu = jnp.mean(x, axis=1, keepdims=True)
    xc = x - mu
    var = jnp.mean(xc * xc, axis=1, keepdims=True)
    hn = xc / jnp.sqrt(var + 1e-5) * par_ref[0:1, :] + par_ref[1:2, :]
    hn_ref[...] = hn
    p2_ref[0] = jnp.dot(hn, whi_ref[...], preferred_element_type=jnp.float32)
    p2_ref[1] = jnp.dot(hn, whj_ref[...], preferred_element_type=jnp.float32)


def _graph_body(xg_ref, l9_ref, wg_ref, par_ref, tab_ref):
    gc = jnp.dot(xg_ref[...], wg_ref[...], preferred_element_type=jnp.float32)
    gc = gc + par_ref[0:1, :]
    l9 = l9_ref[...]
    cols = []
    for i in range(3):
        for j in range(3):
            c = (l9[:, 3 * i + 0:3 * i + 1] * l9[:, 3 * j + 0:3 * j + 1]
                 + l9[:, 3 * i + 1:3 * i + 2] * l9[:, 3 * j + 1:3 * j + 2]
                 + l9[:, 3 * i + 2:3 * i + 3] * l9[:, 3 * j + 2:3 * j + 3])
            cols.append(c)
    pad = jnp.zeros((_G, _WPAD - 128 - 9), jnp.float32)
    tab_ref[...] = jnp.concatenate([gc] + cols + [pad], axis=1)


def _edge_body(hi_ref, hj_ref, g3_ref, fd_ref, tab_ref, par_ref,
               wsin_ref, wcos_ref, we2_ref, out_ref):
    b = hi_ref.shape[0]
    g_row = g3_ref[0]
    iota_c = lax.broadcasted_iota(jnp.int32, (_G, b), 0)
    ot = (iota_c == g_row).astype(jnp.float32)
    tcon = lax.dot_general(ot, tab_ref[...],
                           dimension_numbers=(((0,), (0,)), ((), ())),
                           preferred_element_type=jnp.float32)
    fd = fd_ref[...]
    f0, f1, f2 = fd[:, 0:1], fd[:, 1:2], fd[:, 2:3]
    d = []
    for i in range(3):
        d.append(tcon[:, 128 + 3 * i:129 + 3 * i] * f0
                 + tcon[:, 129 + 3 * i:130 + 3 * i] * f1
                 + tcon[:, 130 + 3 * i:131 + 3 * i] * f2)
    nrm = jnp.sqrt(d[0] * d[0] + d[1] * d[1] + d[2] * d[2]) + 1e-12
    z = hi_ref[...] + hj_ref[...] + tcon[:, 0:128]
    for i in range(3):
        z = z + (d[i] / nrm) * par_ref[i:i + 1, :]
    c48 = lax.broadcasted_iota(jnp.int32, (b, 48), 1)
    xsel = jnp.where(c48 < 16, f0, jnp.where(c48 < 32, f1, f2))
    emb = xsel * par_ref[4:5, 0:48]
    z = z + jnp.dot(jnp.sin(emb), wsin_ref[...],
                    preferred_element_type=jnp.float32)
    z = z + jnp.dot(jnp.cos(emb), wcos_ref[...],
                    preferred_element_type=jnp.float32)
    e1 = z * jax.nn.sigmoid(z)
    y = jnp.dot(e1, we2_ref[...], preferred_element_type=jnp.float32)
    y = y + par_ref[3:4, :]
    e2 = y * jax.nn.sigmoid(y)
    c16 = lax.broadcasted_iota(jnp.int32, (b, 16), 1)
    pad16 = jnp.where(c16 == 0, 1.0, 0.0).astype(jnp.float32)
    out_ref[...] = jnp.concatenate([e2, pad16], axis=1)


def _epilogue_body(p0_ref, p1_ref, h_ref, hn_ref, wn1h_ref, wn1a_ref,
                   wn2_ref, par_ref, out_ref):
    s = p0_ref[0] + p1_ref[0]
    cnt = jnp.maximum(s[:, 128:129], 1.0)
    agg = s[:, 0:128] / cnt
    u = (jnp.dot(hn_ref[...], wn1h_ref[...], preferred_element_type=jnp.float32)
         + jnp.dot(agg, wn1a_ref[...], preferred_element_type=jnp.float32)
         + par_ref[0:1, :])
    u = u * jax.nn.sigmoid(u)
    v = jnp.dot(u, wn2_ref[...], preferred_element_type=jnp.float32)
    v = v + par_ref[1:2, :]
    out_ref[...] = h_ref[...] + v * jax.nn.sigmoid(v)


@functools.lru_cache(maxsize=None)
def _sc_kernels():
    mesh = plsc.VectorSubcoreMesh(core_axis_name="c", subcore_axis_name="s")

    @functools.partial(
        pl.kernel,
        out_type=jax.ShapeDtypeStruct((2 * _E, _H), jnp.float32),
        mesh=mesh,
        scratch_types=[
            pltpu.VMEM((_GCH,), jnp.int32),
            pltpu.VMEM((_GCH, _H), jnp.float32),
            pltpu.SemaphoreType.DMA,
        ],
    )
    def sc_gather(tab_hbm, idx_hbm, out_hbm, idx_v, rows_v, sem):
        wid = lax.axis_index("s") * _NC + lax.axis_index("c")
        base = wid * _GROWS

        @pl.loop(0, _GCHUNKS)
        def _(ci):
            off = base + ci * _GCH
            pltpu.sync_copy(idx_hbm.at[pl.ds(off, _GCH)], idx_v)
            pltpu.async_copy(tab_hbm.at[idx_v], rows_v, sem).wait()
            pltpu.sync_copy(rows_v, out_hbm.at[pl.ds(off, _GCH)])

    @functools.partial(
        pl.kernel,
        out_type=jax.ShapeDtypeStruct((_NC, _N, _WPAD), jnp.float32),
        mesh=mesh,
        compiler_params=pltpu.CompilerParams(use_tc_tiling_on_sc=False),
        scratch_types=[
            pltpu.VMEM((_SCH,), jnp.int32),
            pltpu.VMEM((_SCH, _WPAD), jnp.float32),
            pltpu.VMEM((_ZB, _WPAD), jnp.float32),
            pltpu.VMEM_SHARED((_N, _WPAD), jnp.float32),
            pltpu.SemaphoreType.DMA,
        ],
    )
    def sc_scatter(e2_hbm, src_hbm, out_hbm, idx_v, rows_v, zbuf, acc, sem):
        cid = lax.axis_index("c")
        sid = lax.axis_index("s")
        wid = sid * _NC + cid

        zv = jnp.zeros((16,), jnp.float32)

        @pl.loop(0, _ZB)
        def _(r):
            @pl.loop(0, _WPAD, step=16)
            def _(cc):
                zbuf[r, pl.ds(cc, 16)] = zv

        @pl.loop(sid, _N // _ZB, step=_NS)
        def _(k):
            pltpu.sync_copy(zbuf, acc.at[pl.ds(k * _ZB, _ZB)])

        plsc.subcore_barrier()

        @pl.loop(wid, _SCHUNKS, step=_NW)
        def _(j):
            off = j * _SCH
            pltpu.sync_copy(src_hbm.at[pl.ds(off, _SCH)], idx_v)
            pltpu.sync_copy(e2_hbm.at[pl.ds(off, _SCH)], rows_v)
            pltpu.sync_copy(rows_v, acc.at[idx_v], add=True)

        plsc.subcore_barrier()

        @pl.loop(sid, _N // _ZB, step=_NS)
        def _(k):
            pltpu.sync_copy(acc.at[pl.ds(k * _ZB, _ZB)],
                            out_hbm.at[cid, pl.ds(k * _ZB, _ZB)])

    return sc_gather, sc_scatter


def _run_prologue(h, params, whi_t, whj_t, interpret=False):
    nb = _N // _BN
    return pl.pallas_call(
        _prologue_body,
        grid=(nb,),
        in_specs=[
            pl.BlockSpec((_BN, _H), lambda i: (i, 0)),
            pl.BlockSpec((8, _H), lambda i: (0, 0)),
            pl.BlockSpec((_H, _H), lambda i: (0, 0)),
            pl.BlockSpec((_H, _H), lambda i: (0, 0)),
        ],
        out_specs=[
            pl.BlockSpec((_BN, _H), lambda i: (i, 0)),
            pl.BlockSpec((2, _BN, _H), lambda i: (0, i, 0)),
        ],
        out_shape=[
            jax.ShapeDtypeStruct((_N, _H), jnp.float32),
            jax.ShapeDtypeStruct((2, _N, _H), jnp.float32),
        ],
        interpret=interpret,
    )(h, params, whi_t, whj_t)


def _run_graph(xg, l9, wg, params, interpret=False):
    return pl.pallas_call(
        _graph_body,
        grid=(1,),
        in_specs=[
            pl.BlockSpec((_G, 112), lambda i: (0, 0)),
            pl.BlockSpec((_G, 9), lambda i: (0, 0)),
            pl.BlockSpec((112, _H), lambda i: (0, 0)),
            pl.BlockSpec((8, _H), lambda i: (0, 0)),
        ],
        out_specs=pl.BlockSpec((_G, _WPAD), lambda i: (0, 0)),
        out_shape=jax.ShapeDtypeStruct((_G, _WPAD), jnp.float32),
        interpret=interpret,
    )(xg, l9, wg, params)


def _run_edge(hj_all, g3, fd, tab, params, wsin, wcos, we2_t, interpret=False):
    nb = _E // _BE
    return pl.pallas_call(
        _edge_body,
        grid=(nb,),
        in_specs=[
            pl.BlockSpec((_BE, _H), lambda i: (i, 0)),
            pl.BlockSpec((_BE, _H), lambda i: (i + _E // _BE, 0)),
            pl.BlockSpec((1, 1, _BE), lambda i: (i, 0, 0)),
            pl.BlockSpec((_BE, 3), lambda i: (i, 0)),
            pl.BlockSpec((_G, _WPAD), lambda i: (0, 0)),
            pl.BlockSpec((8, _H), lambda i: (0, 0)),
            pl.BlockSpec((48, _H), lambda i: (0, 0)),
            pl.BlockSpec((48, _H), lambda i: (0, 0)),
            pl.BlockSpec((_H, _H), lambda i: (0, 0)),
        ],
        out_specs=pl.BlockSpec((_BE, _WPAD), lambda i: (i, 0)),
        out_shape=jax.ShapeDtypeStruct((_E, _WPAD), jnp.float32),
        interpret=interpret,
    )(hj_all, hj_all, g3, fd, tab, params, wsin, wcos, we2_t)


def _run_epilogue(partials, h, hn, wn1h_t, wn1a_t, wn2_t, params,
                  interpret=False):
    nb = _N // _BN
    return pl.pallas_call(
        _epilogue_body,
        grid=(nb,),
        in_specs=[
            pl.BlockSpec((1, _BN, _WPAD), lambda i: (0, i, 0)),
            pl.BlockSpec((1, _BN, _WPAD), lambda i: (1, i, 0)),
            pl.BlockSpec((_BN, _H), lambda i: (i, 0)),
            pl.BlockSpec((_BN, _H), lambda i: (i, 0)),
            pl.BlockSpec((_H, _H), lambda i: (0, 0)),
            pl.BlockSpec((_H, _H), lambda i: (0, 0)),
            pl.BlockSpec((_H, _H), lambda i: (0, 0)),
            pl.BlockSpec((8, _H), lambda i: (0, 0)),
        ],
        out_specs=pl.BlockSpec((_BN, _H), lambda i: (i, 0)),
        out_shape=jax.ShapeDtypeStruct((_N, _H), jnp.float32),
        interpret=interpret,
    )(partials, partials, h, hn, wn1h_t, wn1a_t, wn2_t, params)


def kernel(h, lattice_flat, non_zscored_lattice, frac_diff, num_atoms_one_hot,
           ln_gamma, ln_beta, W_na, W_e1, b_e1, W_e2, b_e2, W_n1, b_n1,
           W_n2, b_n2, edge_index, edge2graph):
    f32 = jnp.float32
    zero_row = jnp.zeros((1, _H), f32)

    whi_t = W_e1[:, 3:131].T
    whj_t = W_e1[:, 131:259].T
    wlfe_t = W_e1[:, 259:265].T
    wsin = W_e1[:, 265:313].T
    wcos = W_e1[:, 313:361].T
    m_na = W_na.T @ W_e1[:, 361:489].T
    wg = jnp.concatenate([wlfe_t, m_na, jnp.zeros((6, _H), f32)], axis=0)

    par_pro = jnp.concatenate(
        [ln_gamma[None, :], ln_beta[None, :], jnp.tile(zero_row, (6, 1))],
        axis=0)
    freqs = 2.0 * math.pi * jnp.arange(_NF, dtype=f32)
    freqs48 = jnp.tile(freqs, 3)
    par_edge = jnp.concatenate(
        [W_e1[:, 0:3].T,
         b_e2[None, :],
         jnp.pad(freqs48, (0, _H - 48))[None, :],
         jnp.tile(zero_row, (3, 1))], axis=0)
    par_gr = jnp.concatenate([b_e1[None, :], jnp.tile(zero_row, (7, 1))],
                             axis=0)
    par_epi = jnp.concatenate(
        [b_n1[None, :], b_n2[None, :], jnp.tile(zero_row, (6, 1))], axis=0)

    xg = jnp.concatenate(
        [lattice_flat, num_atoms_one_hot, jnp.zeros((_G, 6), f32)], axis=1)
    l9 = non_zscored_lattice.reshape(_G, 9)

    hn, p2 = _run_prologue(h, par_pro, whi_t, whj_t)
    tab = _run_graph(xg, l9, wg, par_gr)

    sc_gather, sc_scatter = _sc_kernels()
    src = edge_index[0].astype(jnp.int32)
    dst = edge_index[1].astype(jnp.int32)
    idx_all = jnp.concatenate([src, dst + _N])
    hj_all = sc_gather(p2.reshape(2 * _N, _H), idx_all)

    g3 = edge2graph.astype(jnp.int32).reshape(_E // _BE, 1, _BE)
    e2 = _run_edge(hj_all, g3, frac_diff, tab, par_edge, wsin, wcos, W_e2.T)

    partials = sc_scatter(e2, src)

    out = _run_epilogue(partials, h, hn, W_n1[:, :_H].T, W_n1[:, _H:].T,
                        W_n2.T, par_epi)
    return out


# speedup vs baseline: 3.4882x; 1.4052x over previous
import functools
import math

import jax
import jax.numpy as jnp
from jax import lax
from jax.experimental import pallas as pl
from jax.experimental.pallas import tpu as pltpu
from jax.experimental.pallas import tpu_sc as plsc

_N = 10000
_E = 160000
_G = 256
_H = 128
_NF = 16

_BN = 1000
_BE = 1280
_TW = 144

_NC = 2
_NS = 16
_NW = _NC * _NS
_GCH = 80
_GROWS = 2 * _E // _NW
_GCHUNKS = _GROWS // _GCH
_SCH = 128
_SCHUNKS = _E // _SCH
_ZB = 40

_BF = jnp.bfloat16
_F32 = jnp.float32


def _prologue_body(h_ref, par_ref, whi_ref, whj_ref, hn_ref, p2_ref):
    x = h_ref[...]
    mu = jnp.mean(x, axis=1, keepdims=True)
    xc = x - mu
    var = jnp.mean(xc * xc, axis=1, keepdims=True)
    hn = xc / jnp.sqrt(var + 1e-5) * par_ref[0:1, :] + par_ref[1:2, :]
    hn_ref[...] = hn
    p2_ref[0] = jnp.dot(hn, whi_ref[...], preferred_element_type=_F32)
    p2_ref[1] = jnp.dot(hn, whj_ref[...], preferred_element_type=_F32)


def _graph_body(xg_ref, l9_ref, wg_ref, par_ref, tab_ref):
    gc = jnp.dot(xg_ref[...], wg_ref[...], preferred_element_type=_F32)
    gc = gc + par_ref[0:1, :]
    l9 = l9_ref[...]
    cols = []
    for i in range(3):
        for j in range(3):
            c = (l9[:, 3 * i + 0:3 * i + 1] * l9[:, 3 * j + 0:3 * j + 1]
                 + l9[:, 3 * i + 1:3 * i + 2] * l9[:, 3 * j + 1:3 * j + 2]
                 + l9[:, 3 * i + 2:3 * i + 3] * l9[:, 3 * j + 2:3 * j + 3])
            cols.append(c)
    pad = jnp.zeros((_G, _TW - 128 - 9), _F32)
    tab_ref[...] = jnp.concatenate([gc] + cols + [pad], axis=1).astype(_BF)


def _edge_body(hi_ref, hj_ref, g3_ref, fd_ref, tab_ref, par_ref, sel_ref,
               r9_ref, c9_ref, on8_ref, wud_ref, wsin_ref, wcos_ref, we2_ref,
               out_ref):
    b = hi_ref.shape[0]
    g_row = g3_ref[0]
    iota_c = lax.broadcasted_iota(jnp.int32, (_G, b), 0)
    ot = (iota_c == g_row).astype(_BF)
    tcon = lax.dot_general(ot, tab_ref[...],
                           dimension_numbers=(((0,), (0,)), ((), ())),
                           preferred_element_type=_F32)
    fd = fd_ref[...]
    fdh = fd.astype(_BF)
    fdl = (fd - fdh.astype(_F32)).astype(_BF)
    fd9 = jnp.dot(fdh, r9_ref[...], preferred_element_type=_F32)
    x9 = (tcon[:, 128:144] * fd9).astype(_BF)
    dots = jnp.dot(x9, c9_ref[...], preferred_element_type=_F32)
    q = (dots * dots).astype(_BF)
    s2 = jnp.dot(q, on8_ref[...], preferred_element_type=_F32)
    ud = (dots / (jnp.sqrt(s2) + 1e-12)).astype(_BF)
    z = hi_ref[...] + hj_ref[...] + tcon[:, 0:128]
    z = z + jnp.dot(ud, wud_ref[...], preferred_element_type=_F32)
    fdx = (jnp.dot(fdh, sel_ref[...], preferred_element_type=_F32)
           + jnp.dot(fdl, sel_ref[...], preferred_element_type=_F32))
    emb = fdx * par_ref[4:5, 0:48]
    z = z + jnp.dot(jnp.sin(emb).astype(_BF), wsin_ref[...],
                    preferred_element_type=_F32)
    z = z + jnp.dot(jnp.cos(emb).astype(_BF), wcos_ref[...],
                    preferred_element_type=_F32)
    e1 = (z * jax.nn.sigmoid(z)).astype(_BF)
    y = jnp.dot(e1, we2_ref[...], preferred_element_type=_F32)
    y = y + par_ref[3:4, :]
    out_ref[...] = y * jax.nn.sigmoid(y)


def _epilogue_body(p0_ref, p1_ref, c0_ref, c1_ref, h_ref, hn_ref, e0_ref,
                   wn1h_ref, wn1a_ref, wn2_ref, par_ref, out_ref):
    c16 = jnp.maximum(c0_ref[0] + c1_ref[0], 1.0)
    r16 = (1.0 / c16).astype(_BF)
    rbc = jnp.dot(r16, e0_ref[...], preferred_element_type=_F32)
    agg = (p0_ref[0] + p1_ref[0]) * rbc
    u = (jnp.dot(hn_ref[...], wn1h_ref[...], preferred_element_type=_F32)
         + jnp.dot(agg, wn1a_ref[...], preferred_element_type=_F32)
         + par_ref[0:1, :])
    u = u * jax.nn.sigmoid(u)
    v = jnp.dot(u, wn2_ref[...], preferred_element_type=_F32)
    v = v + par_ref[1:2, :]
    out_ref[...] = h_ref[...] + v * jax.nn.sigmoid(v)


@functools.lru_cache(maxsize=None)
def _sc_kernels():
    mesh = plsc.VectorSubcoreMesh(core_axis_name="c", subcore_axis_name="s")

    @functools.partial(
        pl.kernel,
        out_type=jax.ShapeDtypeStruct((_NC, _N, 16), _F32),
        mesh=mesh,
        compiler_params=pltpu.CompilerParams(use_tc_tiling_on_sc=False),
        scratch_types=[
            pltpu.VMEM((_SCH,), jnp.int32),
            pltpu.VMEM((_SCH, 16), _F32),
            pltpu.VMEM((_ZB, 16), _F32),
            pltpu.VMEM_SHARED((_N, 16), _F32),
            pltpu.SemaphoreType.DMA,
        ],
    )
    def sc_counts(src_hbm, out_hbm, idx_v, crow, zbuf, acc, sem):
        cid = lax.axis_index("c")
        sid = lax.axis_index("s")
        wid = sid * _NC + cid

        zv = jnp.zeros((16,), _F32)
        ones0 = jnp.where(lax.iota(jnp.int32, 16) == 0, 1.0, 0.0)

        @pl.loop(0, _SCH)
        def _(r):
            crow[r] = ones0

        @pl.loop(0, _ZB)
        def _(r):
            zbuf[r] = zv

        @pl.loop(sid, _N // _ZB, step=_NS)
        def _(k):
            pltpu.sync_copy(zbuf, acc.at[pl.ds(k * _ZB, _ZB)])

        plsc.subcore_barrier()

        @pl.loop(wid, _SCHUNKS, step=_NW)
        def _(j):
            pltpu.sync_copy(src_hbm.at[pl.ds(j * _SCH, _SCH)], idx_v)
            pltpu.sync_copy(crow, acc.at[idx_v], add=True)

        plsc.subcore_barrier()

        @pl.loop(sid, _N // _ZB, step=_NS)
        def _(k):
            pltpu.sync_copy(acc.at[pl.ds(k * _ZB, _ZB)],
                            out_hbm.at[cid, pl.ds(k * _ZB, _ZB)])

    @functools.partial(
        pl.kernel,
        out_type=jax.ShapeDtypeStruct((2 * _E, _H), _F32),
        mesh=mesh,
        scratch_types=[
            pltpu.VMEM((_GCH,), jnp.int32),
            pltpu.VMEM((_GCH, _H), _F32),
            pltpu.SemaphoreType.DMA,
        ],
    )
    def sc_gather(tab_hbm, idx_hbm, out_hbm, idx_v, rows_v, sem):
        wid = lax.axis_index("s") * _NC + lax.axis_index("c")
        base = wid * _GROWS

        @pl.loop(0, _GCHUNKS)
        def _(ci):
            off = base + ci * _GCH
            pltpu.sync_copy(idx_hbm.at[pl.ds(off, _GCH)], idx_v)
            pltpu.async_copy(tab_hbm.at[idx_v], rows_v, sem).wait()
            pltpu.sync_copy(rows_v, out_hbm.at[pl.ds(off, _GCH)])

    @functools.partial(
        pl.kernel,
        out_type=jax.ShapeDtypeStruct((_NC, _N, _H), _F32),
        mesh=mesh,
        scratch_types=[
            pltpu.VMEM((_SCH,), jnp.int32),
            pltpu.VMEM((_SCH, _H), _F32),
            pltpu.VMEM((_ZB, _H), _F32),
            pltpu.VMEM_SHARED((_N, _H), _F32),
            pltpu.SemaphoreType.DMA,
        ],
    )
    def sc_scatter(e2_hbm, src_hbm, out_hbm, idx_v, rows_v, zbuf, acc, sem):
        cid = lax.axis_index("c")
        sid = lax.axis_index("s")
        wid = sid * _NC + cid

        zv = jnp.zeros((16,), _F32)

        @pl.loop(0, _ZB)
        def _(r):
            @pl.loop(0, _H, step=16)
            def _(cc):
                zbuf[r, pl.ds(cc, 16)] = zv

        @pl.loop(sid, _N // _ZB, step=_NS)
        def _(k):
            pltpu.sync_copy(zbuf, acc.at[pl.ds(k * _ZB, _ZB)])

        plsc.subcore_barrier()

        @pl.loop(wid, _SCHUNKS, step=_NW)
        def _(j):
            off = j * _SCH
            pltpu.sync_copy(src_hbm.at[pl.ds(off, _SCH)], idx_v)
            pltpu.sync_copy(e2_hbm.at[pl.ds(off, _SCH)], rows_v)
            pltpu.sync_copy(rows_v, acc.at[idx_v], add=True)

        plsc.subcore_barrier()

        @pl.loop(sid, _N // _ZB, step=_NS)
        def _(k):
            pltpu.sync_copy(acc.at[pl.ds(k * _ZB, _ZB)],
                            out_hbm.at[cid, pl.ds(k * _ZB, _ZB)])

    return sc_counts, sc_gather, sc_scatter


def _run_prologue(h, params, whi_t, whj_t, interpret=False):
    nb = _N // _BN
    return pl.pallas_call(
        _prologue_body,
        grid=(nb,),
        in_specs=[
            pl.BlockSpec((_BN, _H), lambda i: (i, 0)),
            pl.BlockSpec((8, _H), lambda i: (0, 0)),
            pl.BlockSpec((_H, _H), lambda i: (0, 0)),
            pl.BlockSpec((_H, _H), lambda i: (0, 0)),
        ],
        out_specs=[
            pl.BlockSpec((_BN, _H), lambda i: (i, 0)),
            pl.BlockSpec((2, _BN, _H), lambda i: (0, i, 0)),
        ],
        out_shape=[
            jax.ShapeDtypeStruct((_N, _H), _F32),
            jax.ShapeDtypeStruct((2, _N, _H), _F32),
        ],
        interpret=interpret,
    )(h, params, whi_t, whj_t)


def _run_graph(xg, l9, wg, params, interpret=False):
    return pl.pallas_call(
        _graph_body,
        grid=(1,),
        in_specs=[
            pl.BlockSpec((_G, 112), lambda i: (0, 0)),
            pl.BlockSpec((_G, 9), lambda i: (0, 0)),
            pl.BlockSpec((112, _H), lambda i: (0, 0)),
            pl.BlockSpec((8, _H), lambda i: (0, 0)),
        ],
        out_specs=pl.BlockSpec((_G, _TW), lambda i: (0, 0)),
        out_shape=jax.ShapeDtypeStruct((_G, _TW), _BF),
        interpret=interpret,
    )(xg, l9, wg, params)


def _run_edge(hj_all, g3, fd8, tab, params, sel, r9, c9, on8, wud, wsin,
              wcos, we2, interpret=False):
    nb = _E // _BE
    return pl.pallas_call(
        _edge_body,
        grid=(nb,),
        in_specs=[
            pl.BlockSpec((_BE, _H), lambda i: (i, 0)),
            pl.BlockSpec((_BE, _H), lambda i: (i + _E // _BE, 0)),
            pl.BlockSpec((1, 1, _BE), lambda i: (i, 0, 0)),
            pl.BlockSpec((_BE, 8), lambda i: (i, 0)),
            pl.BlockSpec((_G, _TW), lambda i: (0, 0)),
            pl.BlockSpec((8, _H), lambda i: (0, 0)),
            pl.BlockSpec((8, 48), lambda i: (0, 0)),
            pl.BlockSpec((8, 16), lambda i: (0, 0)),
            pl.BlockSpec((16, 8), lambda i: (0, 0)),
            pl.BlockSpec((8, 8), lambda i: (0, 0)),
            pl.BlockSpec((8, _H), lambda i: (0, 0)),
            pl.BlockSpec((48, _H), lambda i: (0, 0)),
            pl.BlockSpec((48, _H), lambda i: (0, 0)),
            pl.BlockSpec((_H, _H), lambda i: (0, 0)),
        ],
        out_specs=pl.BlockSpec((_BE, _H), lambda i: (i, 0)),
        out_shape=jax.ShapeDtypeStruct((_E, _H), _F32),
        interpret=interpret,
    )(hj_all, hj_all, g3, fd8, tab, params, sel, r9, c9, on8, wud, wsin,
      wcos, we2)


def _run_epilogue(partials, cnts, h, hn, e0, wn1h_t, wn1a_t, wn2_t, params,
                  interpret=False):
    nb = _N // _BN
    return pl.pallas_call(
        _epilogue_body,
        grid=(nb,),
        in_specs=[
            pl.BlockSpec((1, _BN, _H), lambda i: (0, i, 0)),
            pl.BlockSpec((1, _BN, _H), lambda i: (1, i, 0)),
            pl.BlockSpec((1, _BN, 16), lambda i: (0, i, 0)),
            pl.BlockSpec((1, _BN, 16), lambda i: (1, i, 0)),
            pl.BlockSpec((_BN, _H), lambda i: (i, 0)),
            pl.BlockSpec((_BN, _H), lambda i: (i, 0)),
            pl.BlockSpec((16, _H), lambda i: (0, 0)),
            pl.BlockSpec((_H, _H), lambda i: (0, 0)),
            pl.BlockSpec((_H, _H), lambda i: (0, 0)),
            pl.BlockSpec((_H, _H), lambda i: (0, 0)),
            pl.BlockSpec((8, _H), lambda i: (0, 0)),
        ],
        out_specs=pl.BlockSpec((_BN, _H), lambda i: (i, 0)),
        out_shape=jax.ShapeDtypeStruct((_N, _H), _F32),
        interpret=interpret,
    )(partials, partials, cnts, cnts, h, hn, e0, wn1h_t, wn1a_t, wn2_t,
      params)


def kernel(h, lattice_flat, non_zscored_lattice, frac_diff, num_atoms_one_hot,
           ln_gamma, ln_beta, W_na, W_e1, b_e1, W_e2, b_e2, W_n1, b_n1,
           W_n2, b_n2, edge_index, edge2graph):
    import numpy as np
    zero_row = jnp.zeros((1, _H), _F32)

    whi_t = W_e1[:, 3:131].T
    whj_t = W_e1[:, 131:259].T
    wlfe_t = W_e1[:, 259:265].T
    wsin = W_e1[:, 265:313].T.astype(_BF)
    wcos = W_e1[:, 313:361].T.astype(_BF)
    m_na = W_na.T @ W_e1[:, 361:489].T
    wg = jnp.concatenate([wlfe_t, m_na, jnp.zeros((6, _H), _F32)], axis=0)
    wud = jnp.concatenate([W_e1[:, 0:3].T, jnp.zeros((5, _H), _F32)],
                          axis=0).astype(_BF)

    par_pro = jnp.concatenate(
        [ln_gamma[None, :], ln_beta[None, :], jnp.tile(zero_row, (6, 1))],
        axis=0)
    freqs = 2.0 * math.pi * jnp.arange(_NF, dtype=_F32)
    freqs48 = jnp.tile(freqs, 3)
    par_edge = jnp.concatenate(
        [jnp.tile(zero_row, (3, 1)),
         b_e2[None, :],
         jnp.pad(freqs48, (0, _H - 48))[None, :],
         jnp.tile(zero_row, (3, 1))], axis=0)
    par_gr = jnp.concatenate([b_e1[None, :], jnp.tile(zero_row, (7, 1))],
                             axis=0)
    par_epi = jnp.concatenate(
        [b_n1[None, :], b_n2[None, :], jnp.tile(zero_row, (6, 1))], axis=0)

    sel_np = np.zeros((8, 48), np.float32)
    for j in range(3):
        sel_np[j, j * 16:(j + 1) * 16] = 1.0
    r9_np = np.zeros((8, 16), np.float32)
    c9_np = np.zeros((16, 8), np.float32)
    for i in range(3):
        for j in range(3):
            r9_np[j, 3 * i + j] = 1.0
            c9_np[3 * i + j, i] = 1.0
    sel = jnp.asarray(sel_np, _BF)
    r9 = jnp.asarray(r9_np, _BF)
    c9 = jnp.asarray(c9_np, _BF)
    on8 = jnp.ones((8, 8), _BF)
    e0_np = np.zeros((16, _H), np.float32)
    e0_np[0, :] = 1.0
    e0 = jnp.asarray(e0_np, _BF)

    xg = jnp.concatenate(
        [lattice_flat, num_atoms_one_hot, jnp.zeros((_G, 6), _F32)], axis=1)
    l9 = non_zscored_lattice.reshape(_G, 9)
    fd8 = jnp.pad(frac_diff, ((0, 0), (0, 5)))

    sc_counts, sc_gather, sc_scatter = _sc_kernels()
    src = edge_index[0].astype(jnp.int32)
    dst = edge_index[1].astype(jnp.int32)

    cnts = sc_counts(src)

    hn, p2 = _run_prologue(h, par_pro, whi_t, whj_t)
    tab = _run_graph(xg, l9, wg, par_gr)

    idx_all = jnp.concatenate([src, dst + _N])
    hj_all = sc_gather(p2.reshape(2 * _N, _H), idx_all)

    g3 = edge2graph.astype(jnp.int32).reshape(_E // _BE, 1, _BE)
    e2 = _run_edge(hj_all, g3, fd8, tab, par_edge, sel, r9, c9, on8, wud,
                   wsin, wcos, W_e2.T.astype(_BF))

    partials = sc_scatter(e2, src)

    out = _run_epilogue(partials, cnts, h, hn, e0, W_n1[:, :_H].T,
                        W_n1[:, _H:].T, W_n2.T, par_epi)
    return out


# speedup vs baseline: 7.6390x; 2.1900x over previous
import functools
import math

import jax
import jax.numpy as jnp
from jax import lax
from jax.experimental import pallas as pl
from jax.experimental.pallas import tpu as pltpu
from jax.experimental.pallas import tpu_sc as plsc

_N = 10000
_E = 160000
_G = 256
_H = 128
_NF = 16

_BN = 1000
_BE = 1280
_S = 5
_ES = _E // _S

_NC = 2
_NS = 16
_NW = _NC * _NS
_GCH = 40
_GROWS = 2 * _ES // _NW
_GCHUNKS = _GROWS // _GCH
_SCH = 128
_SCHUNKS = _ES // _SCH
_CCHUNKS = _E // _SCH
_ZB = 40

_BF = jnp.bfloat16
_F32 = jnp.float32


def _prologue_body(h_ref, par_ref, whi_ref, whj_ref, hn_ref, p2_ref):
    x = h_ref[...]
    mu = jnp.mean(x, axis=1, keepdims=True)
    xc = x - mu
    var = jnp.mean(xc * xc, axis=1, keepdims=True)
    hn = xc / jnp.sqrt(var + 1e-5) * par_ref[0:1, :] + par_ref[1:2, :]
    hn_ref[...] = hn
    ps = jnp.dot(hn, whi_ref[...], preferred_element_type=_F32)
    pd = jnp.dot(hn, whj_ref[...], preferred_element_type=_F32)
    us = lax.bitcast_convert_type(ps, jnp.uint32)
    us = (us + jnp.uint32(0x8000)) & jnp.uint32(0xFFFF0000)
    ud = lax.bitcast_convert_type(pd, jnp.uint32)
    ud = (ud + jnp.uint32(0x8000)) >> 16
    p2_ref[...] = lax.bitcast_convert_type(us | ud, jnp.int32)


def _graph_body(xg_ref, l9_ref, wg_ref, par_ref, tab_ref, tab9_ref):
    gc = jnp.dot(xg_ref[...], wg_ref[...], preferred_element_type=_F32)
    gc = gc + par_ref[0:1, :]
    tab_ref[...] = gc.astype(_BF)
    l9 = l9_ref[...]
    cols = []
    for i in range(3):
        for j in range(3):
            c = (l9[:, 3 * i + 0:3 * i + 1] * l9[:, 3 * j + 0:3 * j + 1]
                 + l9[:, 3 * i + 1:3 * i + 2] * l9[:, 3 * j + 1:3 * j + 2]
                 + l9[:, 3 * i + 2:3 * i + 3] * l9[:, 3 * j + 2:3 * j + 3])
            cols.append(c)
    pad = jnp.zeros((_G, 7), _F32)
    tab9_ref[...] = jnp.concatenate(cols + [pad], axis=1).astype(_BF)


_TWOPI = 2.0 * math.pi


def _edge_body(hi_ref, hj_ref, aux_ref, tab_ref, tab9_ref, par_ref, c9_ref,
               o83_ref, wud_ref, wsc_ref, we2_ref, out_ref, scr_ref):
    b = hi_ref.shape[0]
    a8 = aux_ref[...]
    g_row = a8[3:4, :].astype(jnp.int32)
    iota_c = lax.broadcasted_iota(jnp.int32, (_G, b), 0)
    ot = (iota_c == g_row).astype(_BF)
    tcon = lax.dot_general(ot, tab_ref[...],
                           dimension_numbers=(((0,), (0,)), ((), ())),
                           preferred_element_type=_F32)
    tc9 = lax.dot_general(tab9_ref[...], ot,
                          dimension_numbers=(((0,), (0,)), ((), ())),
                          preferred_element_type=_F32)
    fd3 = a8[0:3, :]
    fd9t = jnp.concatenate([fd3, fd3, fd3, jnp.zeros((7, b), _F32)], axis=0)
    x9t = (tc9 * fd9t).astype(_BF)
    dots = lax.dot_general(c9_ref[...], x9t,
                           dimension_numbers=(((0,), (0,)), ((), ())),
                           preferred_element_type=_F32)
    q = (dots * dots).astype(_BF)
    s2 = lax.dot_general(o83_ref[...], q,
                         dimension_numbers=(((0,), (0,)), ((), ())),
                         preferred_element_type=_F32)
    ud = (dots / (jnp.sqrt(s2) + 1e-12)).astype(_BF)
    ui = lax.bitcast_convert_type(hi_ref[...], jnp.uint32)
    uj = lax.bitcast_convert_type(hj_ref[...], jnp.uint32)
    hi = lax.bitcast_convert_type(ui & jnp.uint32(0xFFFF0000), _F32)
    hj = lax.bitcast_convert_type(uj << 16, _F32)
    z = hi + hj + tcon
    z = z + lax.dot_general(ud, wud_ref[...],
                            dimension_numbers=(((0,), (0,)), ((), ())),
                            preferred_element_type=_F32)
    s1 = jnp.sin(a8 * _TWOPI)
    c1 = jnp.cos(a8 * _TWOPI)
    sk, ck = s1, c1
    scr_ref[0:8, :] = s1.astype(_BF)
    scr_ref[8:16, :] = c1.astype(_BF)
    for k in range(2, _NF):
        sk, ck = sk * c1 + ck * s1, ck * c1 - sk * s1
        g = 16 * (k - 1)
        scr_ref[g:g + 8, :] = sk.astype(_BF)
        scr_ref[g + 8:g + 16, :] = ck.astype(_BF)
    z = z + lax.dot_general(scr_ref[...], wsc_ref[...],
                            dimension_numbers=(((0,), (0,)), ((), ())),
                            preferred_element_type=_F32)
    e1 = (z * jax.nn.sigmoid(z)).astype(_BF)
    y = jnp.dot(e1, we2_ref[...], preferred_element_type=_F32)
    y = y + par_ref[3:4, :]
    out_ref[...] = y * jax.nn.sigmoid(y)


def _epilogue_body(*refs):
    p_refs = refs[0:2 * _S]
    (c0_ref, c1_ref, h_ref, hn_ref, e0_ref, wn1h_ref, wn1a_ref, wn2_ref,
     par_ref, out_ref) = refs[2 * _S:]
    c16 = jnp.maximum(c0_ref[0] + c1_ref[0], 1.0)
    r16 = (1.0 / c16).astype(_BF)
    rbc = jnp.dot(r16, e0_ref[...], preferred_element_type=_F32)
    psum = p_refs[0][0]
    for pr in p_refs[1:]:
        psum = psum + pr[0]
    agg = psum * rbc
    u = (jnp.dot(hn_ref[...], wn1h_ref[...], preferred_element_type=_F32)
         + jnp.dot(agg, wn1a_ref[...], preferred_element_type=_F32)
         + par_ref[0:1, :])
    u = u * jax.nn.sigmoid(u)
    v = jnp.dot(u, wn2_ref[...], preferred_element_type=_F32)
    v = v + par_ref[1:2, :]
    out_ref[...] = h_ref[...] + v * jax.nn.sigmoid(v)


@functools.lru_cache(maxsize=None)
def _sc_kernels():
    mesh = plsc.VectorSubcoreMesh(core_axis_name="c", subcore_axis_name="s")

    @functools.partial(
        pl.kernel,
        out_type=jax.ShapeDtypeStruct((_NC, _N, 16), _F32),
        mesh=mesh,
        compiler_params=pltpu.CompilerParams(use_tc_tiling_on_sc=False),
        scratch_types=[
            pltpu.VMEM((40, _SCH), jnp.int32),
            pltpu.VMEM((_SCH, 16), _F32),
            pltpu.VMEM((_ZB, 16), _F32),
            pltpu.VMEM_SHARED((_N, 16), _F32),
            pltpu.SemaphoreType.DMA,
        ],
    )
    def sc_counts(src_hbm, out_hbm, idx2d, crow, zbuf, acc, sem):
        cid = lax.axis_index("c")
        sid = lax.axis_index("s")
        wid = sid * _NC + cid

        nch = jnp.where(wid < 2, 40, 39)
        start = wid * 39 + jnp.minimum(wid, 2)
        start8 = jnp.minimum(start, _CCHUNKS - 40)
        off0 = start - start8

        zv = jnp.zeros((16,), _F32)
        ones0 = jnp.where(lax.iota(jnp.int32, 16) == 0, 1.0, 0.0)

        pltpu.async_copy(src_hbm.at[pl.ds(start8, 40)], idx2d, sem)

        @pl.loop(0, _SCH)
        def _(r):
            crow[r] = ones0

        @pl.loop(0, _ZB)
        def _(r):
            zbuf[r] = zv

        @pl.loop(sid, _N // _ZB, step=_NS)
        def _(k):
            pltpu.sync_copy(zbuf, acc.at[pl.ds(k * _ZB, _ZB)])

        pltpu.make_async_copy(src_hbm.at[pl.ds(start8, 40)], idx2d,
                              sem).wait()
        plsc.subcore_barrier()

        @pl.loop(0, 40)
        def _(k):
            @pl.when(k < nch)
            def _():
                pltpu.sync_copy(crow, acc.at[idx2d.at[off0 + k]], add=True)

        plsc.subcore_barrier()

        @pl.loop(sid, _N // _ZB, step=_NS)
        def _(k):
            pltpu.sync_copy(acc.at[pl.ds(k * _ZB, _ZB)],
                            out_hbm.at[cid, pl.ds(k * _ZB, _ZB)])

    @functools.partial(
        pl.kernel,
        out_type=jax.ShapeDtypeStruct((2 * _ES, _H), jnp.int32),
        mesh=mesh,
        scratch_types=[
            pltpu.VMEM((_GROWS,), jnp.int32),
            pltpu.VMEM((_GCH, _H), jnp.int32),
            pltpu.VMEM((_GCH, _H), jnp.int32),
            pltpu.SemaphoreType.DMA,
            pltpu.SemaphoreType.DMA,
            pltpu.SemaphoreType.DMA,
            pltpu.SemaphoreType.DMA,
        ],
    )
    def sc_gather(tab_hbm, idx_hbm, out_hbm, idxall, rows0, rows1,
                  gs0, gs1, ws0, ws1):
        wid = lax.axis_index("s") * _NC + lax.axis_index("c")
        base = wid * _GROWS
        rowsv = (rows0, rows1)
        gs = (gs0, gs1)
        ws = (ws0, ws1)

        pltpu.sync_copy(idx_hbm.at[pl.ds(base, _GROWS)], idxall)

        def issue(c, b):
            pltpu.async_copy(tab_hbm.at[idxall.at[pl.ds(c * _GCH, _GCH)]],
                             rowsv[b], gs[b])

        issue(0, 0)
        issue(1, 1)

        @pl.loop(0, _GCHUNKS - 2, step=2)
        def _(c):
            for b in range(2):
                cc = c + b
                dst = out_hbm.at[pl.ds(base + cc * _GCH, _GCH)]
                pltpu.make_async_copy(tab_hbm.at[idxall.at[pl.ds(0, _GCH)]],
                                      rowsv[b], gs[b]).wait()
                pltpu.async_copy(rowsv[b], dst, ws[b])
                pltpu.make_async_copy(rowsv[b], dst, ws[b]).wait()
                issue(cc + 2, b)

        for b in range(2):
            cc = _GCHUNKS - 2 + b
            dst = out_hbm.at[pl.ds(base + cc * _GCH, _GCH)]
            pltpu.make_async_copy(tab_hbm.at[idxall.at[pl.ds(0, _GCH)]],
                                  rowsv[b], gs[b]).wait()
            pltpu.async_copy(rowsv[b], dst, ws[b])
            pltpu.make_async_copy(rowsv[b], dst, ws[b]).wait()

    @functools.partial(
        pl.kernel,
        out_type=jax.ShapeDtypeStruct((_NC, _N, _H), _F32),
        mesh=mesh,
        scratch_types=[
            pltpu.VMEM((_SCH,), jnp.int32),
            pltpu.VMEM((_SCH,), jnp.int32),
            pltpu.VMEM((_SCH, _H), _F32),
            pltpu.VMEM((_SCH, _H), _F32),
            pltpu.VMEM((_ZB, _H), _F32),
            pltpu.VMEM_SHARED((_N, _H), _F32),
            pltpu.SemaphoreType.DMA,
            pltpu.SemaphoreType.DMA,
        ],
    )
    def sc_scatter(e2_hbm, src_hbm, out_hbm, idx0, idx1, rows0, rows1,
                   zbuf, acc, ls0, ls1):
        cid = lax.axis_index("c")
        sid = lax.axis_index("s")
        wid = sid * _NC + cid
        idxv = (idx0, idx1)
        rowsv = (rows0, rows1)
        ls = (ls0, ls1)

        def sissue(j, b):
            off = j * _SCH
            pltpu.async_copy(src_hbm.at[pl.ds(off, _SCH)], idxv[b], ls[b])
            pltpu.async_copy(e2_hbm.at[pl.ds(off, _SCH)], rowsv[b], ls[b])

        def swait(j, b):
            off = j * _SCH
            pltpu.make_async_copy(src_hbm.at[pl.ds(off, _SCH)], idxv[b],
                                  ls[b]).wait()
            pltpu.make_async_copy(e2_hbm.at[pl.ds(off, _SCH)], rowsv[b],
                                  ls[b]).wait()

        zv = jnp.zeros((16,), _F32)

        @pl.loop(0, _ZB)
        def _(r):
            @pl.loop(0, _H, step=16)
            def _(cc):
                zbuf[r, pl.ds(cc, 16)] = zv

        sissue(wid, 0)

        @pl.when(wid + _NW < _SCHUNKS)
        def _():
            sissue(wid + _NW, 1)

        @pl.loop(sid, _N // _ZB, step=_NS)
        def _(k):
            pltpu.sync_copy(zbuf, acc.at[pl.ds(k * _ZB, _ZB)])

        plsc.subcore_barrier()

        @pl.loop(wid, _SCHUNKS, step=2 * _NW)
        def _(j):
            swait(j, 0)
            pltpu.sync_copy(rows0, acc.at[idx0], add=True)

            @pl.when(j + 2 * _NW < _SCHUNKS)
            def _():
                sissue(j + 2 * _NW, 0)

            @pl.when(j + _NW < _SCHUNKS)
            def _():
                swait(j + _NW, 1)
                pltpu.sync_copy(rows1, acc.at[idx1], add=True)

                @pl.when(j + 3 * _NW < _SCHUNKS)
                def _():
                    sissue(j + 3 * _NW, 1)

        plsc.subcore_barrier()

        @pl.loop(sid, _N // _ZB, step=_NS)
        def _(k):
            pltpu.sync_copy(acc.at[pl.ds(k * _ZB, _ZB)],
                            out_hbm.at[cid, pl.ds(k * _ZB, _ZB)])

    return sc_counts, sc_gather, sc_scatter


def _run_prologue(h, params, whi_t, whj_t, interpret=False):
    nb = _N // _BN
    return pl.pallas_call(
        _prologue_body,
        grid=(nb,),
        in_specs=[
            pl.BlockSpec((_BN, _H), lambda i: (i, 0)),
            pl.BlockSpec((8, _H), lambda i: (0, 0)),
            pl.BlockSpec((_H, _H), lambda i: (0, 0)),
            pl.BlockSpec((_H, _H), lambda i: (0, 0)),
        ],
        out_specs=[
            pl.BlockSpec((_BN, _H), lambda i: (i, 0)),
            pl.BlockSpec((_BN, _H), lambda i: (i, 0)),
        ],
        out_shape=[
            jax.ShapeDtypeStruct((_N, _H), _F32),
            jax.ShapeDtypeStruct((_N, _H), jnp.int32),
        ],
        interpret=interpret,
    )(h, params, whi_t, whj_t)


def _run_graph(xg, l9, wg, params, interpret=False):
    return pl.pallas_call(
        _graph_body,
        grid=(1,),
        in_specs=[
            pl.BlockSpec((_G, 112), lambda i: (0, 0)),
            pl.BlockSpec((_G, 9), lambda i: (0, 0)),
            pl.BlockSpec((112, _H), lambda i: (0, 0)),
            pl.BlockSpec((8, _H), lambda i: (0, 0)),
        ],
        out_specs=[
            pl.BlockSpec((_G, _H), lambda i: (0, 0)),
            pl.BlockSpec((_G, 16), lambda i: (0, 0)),
        ],
        out_shape=[
            jax.ShapeDtypeStruct((_G, _H), _BF),
            jax.ShapeDtypeStruct((_G, 16), _BF),
        ],
        interpret=interpret,
    )(xg, l9, wg, params)


def _run_edge(hj_all, aux, tab, tab9, params, c9, o83, wud, wsc, we2,
              slice_idx=0, interpret=False):
    nb = _ES // _BE
    off = slice_idx * nb
    return pl.pallas_call(
        _edge_body,
        grid=(nb,),
        in_specs=[
            pl.BlockSpec((_BE, _H), lambda i: (i, 0)),
            pl.BlockSpec((_BE, _H), lambda i: (i + _ES // _BE, 0)),
            pl.BlockSpec((8, _BE), lambda i: (0, i + off)),
            pl.BlockSpec((_G, _H), lambda i: (0, 0)),
            pl.BlockSpec((_G, 16), lambda i: (0, 0)),
            pl.BlockSpec((8, _H), lambda i: (0, 0)),
            pl.BlockSpec((16, 8), lambda i: (0, 0)),
            pl.BlockSpec((8, 8), lambda i: (0, 0)),
            pl.BlockSpec((8, _H), lambda i: (0, 0)),
            pl.BlockSpec((240, _H), lambda i: (0, 0)),
            pl.BlockSpec((_H, _H), lambda i: (0, 0)),
        ],
        out_specs=pl.BlockSpec((_BE, _H), lambda i: (i, 0)),
        out_shape=jax.ShapeDtypeStruct((_ES, _H), _F32),
        scratch_shapes=[pltpu.VMEM((240, _BE), _BF)],
        interpret=interpret,
    )(hj_all, hj_all, aux, tab, tab9, params, c9, o83, wud, wsc, we2)


def _run_epilogue(partials_list, cnts, h, hn, e0, wn1h_t, wn1a_t, wn2_t,
                  params, interpret=False):
    nb = _N // _BN
    p_specs = []
    p_args = []
    for p in partials_list:
        p_specs.append(pl.BlockSpec((1, _BN, _H), lambda i: (0, i, 0)))
        p_specs.append(pl.BlockSpec((1, _BN, _H), lambda i: (1, i, 0)))
        p_args.extend([p, p])
    return pl.pallas_call(
        _epilogue_body,
        grid=(nb,),
        in_specs=p_specs + [
            pl.BlockSpec((1, _BN, 16), lambda i: (0, i, 0)),
            pl.BlockSpec((1, _BN, 16), lambda i: (1, i, 0)),
            pl.BlockSpec((_BN, _H), lambda i: (i, 0)),
            pl.BlockSpec((_BN, _H), lambda i: (i, 0)),
            pl.BlockSpec((16, _H), lambda i: (0, 0)),
            pl.BlockSpec((_H, _H), lambda i: (0, 0)),
            pl.BlockSpec((_H, _H), lambda i: (0, 0)),
            pl.BlockSpec((_H, _H), lambda i: (0, 0)),
            pl.BlockSpec((8, _H), lambda i: (0, 0)),
        ],
        out_specs=pl.BlockSpec((_BN, _H), lambda i: (i, 0)),
        out_shape=jax.ShapeDtypeStruct((_N, _H), _F32),
        interpret=interpret,
    )(*p_args, cnts, cnts, h, hn, e0, wn1h_t, wn1a_t, wn2_t, params)


def kernel(h, lattice_flat, non_zscored_lattice, frac_diff, num_atoms_one_hot,
           ln_gamma, ln_beta, W_na, W_e1, b_e1, W_e2, b_e2, W_n1, b_n1,
           W_n2, b_n2, edge_index, edge2graph):
    import numpy as np
    zero_row = jnp.zeros((1, _H), _F32)

    whi_t = W_e1[:, 3:131].T
    whj_t = W_e1[:, 131:259].T
    wlfe_t = W_e1[:, 259:265].T
    m_na = W_na.T @ W_e1[:, 361:489].T
    wg = jnp.concatenate([wlfe_t, m_na, jnp.zeros((6, _H), _F32)], axis=0)
    wud = jnp.concatenate([W_e1[:, 0:3].T, jnp.zeros((5, _H), _F32)],
                          axis=0).astype(_BF)

    rows_idx = []
    cols_idx = []
    for k in range(1, _NF):
        for d in range(3):
            rows_idx.append(16 * (k - 1) + d)
            cols_idx.append(265 + d * 16 + k)
            rows_idx.append(16 * (k - 1) + 8 + d)
            cols_idx.append(313 + d * 16 + k)
    wsc = jnp.zeros((240, _H), _F32).at[jnp.asarray(rows_idx)].set(
        W_e1[:, jnp.asarray(cols_idx)].T).astype(_BF)
    b_e1x = b_e1 + W_e1[:, 313] + W_e1[:, 329] + W_e1[:, 345]

    par_pro = jnp.concatenate(
        [ln_gamma[None, :], ln_beta[None, :], jnp.tile(zero_row, (6, 1))],
        axis=0)
    par_edge = jnp.concatenate(
        [jnp.tile(zero_row, (3, 1)),
         b_e2[None, :],
         jnp.tile(zero_row, (4, 1))], axis=0)
    par_gr = jnp.concatenate([b_e1x[None, :], jnp.tile(zero_row, (7, 1))],
                             axis=0)
    par_epi = jnp.concatenate(
        [b_n1[None, :], b_n2[None, :], jnp.tile(zero_row, (6, 1))], axis=0)

    c9_np = np.zeros((16, 8), np.float32)
    for i in range(3):
        for j in range(3):
            c9_np[3 * i + j, i] = 1.0
    c9 = jnp.asarray(c9_np, _BF)
    o83_np = np.zeros((8, 8), np.float32)
    o83_np[0:3, :] = 1.0
    o83 = jnp.asarray(o83_np, _BF)
    e0_np = np.zeros((16, _H), np.float32)
    e0_np[0, :] = 1.0
    e0 = jnp.asarray(e0_np, _BF)

    xg = jnp.concatenate(
        [lattice_flat, num_atoms_one_hot, jnp.zeros((_G, 6), _F32)], axis=1)
    l9 = non_zscored_lattice.reshape(_G, 9)

    sc_counts, sc_gather, sc_scatter = _sc_kernels()
    src = edge_index[0].astype(jnp.int32)
    dst = edge_index[1].astype(jnp.int32)

    cnts = sc_counts(src.reshape(_CCHUNKS, _SCH))

    hn, p2 = _run_prologue(h, par_pro, whi_t, whj_t)
    tab, tab9 = _run_graph(xg, l9, wg, par_gr)

    aux = jnp.concatenate(
        [frac_diff.T, edge2graph.astype(_F32)[None, :],
         jnp.zeros((4, _E), _F32)], axis=0)
    we2t = W_e2.T.astype(_BF)
    partials_list = []
    for s in range(_S):
        src_s = lax.slice(src, (s * _ES,), ((s + 1) * _ES,))
        dst_s = lax.slice(dst, (s * _ES,), ((s + 1) * _ES,))
        idx_s = jnp.concatenate([src_s, dst_s])
        hj_s = sc_gather(p2, idx_s)
        e2_s = _run_edge(hj_s, aux, tab, tab9, par_edge, c9, o83, wud, wsc,
                         we2t, slice_idx=s)
        partials_list.append(sc_scatter(e2_s, src_s))

    out = _run_epilogue(partials_list, cnts, h, hn, e0, W_n1[:, :_H].T,
                        W_n1[:, _H:].T, W_n2.T, par_epi)
    return out
